# baseline (speedup 1.0000x reference)
import functools

import jax
import jax.numpy as jnp
from jax import lax
from jax.experimental import pallas as pl
from jax.experimental.pallas import tpu as pltpu
from jax.experimental.pallas import tpu_sc as plsc

N = 10000
D = 128
NC = 2
NS = 16
NW = NC * NS
CHUNK = 128
SEG_CPT = 16
C0_FRAC = 0.9
SLAB = 632
LAST = N - (NS - 1) * SLAB


@functools.lru_cache(maxsize=None)
def _make_seg_kernel(cpt):
    mesh = plsc.VectorSubcoreMesh(core_axis_name="c", subcore_axis_name="s")
    blk_t = pltpu.VMEM((4, 2, CHUNK), jnp.int32)
    row_t = pltpu.VMEM((CHUNK, D), jnp.float32)

    @functools.partial(
        pl.kernel,
        mesh=mesh,
        out_type=jax.ShapeDtypeStruct((N, D), jnp.float32),
        scratch_types=(
            [pltpu.VMEM_SHARED((N + 8, D), jnp.float32)]
            + [blk_t] * 2
            + [row_t] * 2
            + [pltpu.SemaphoreType.DMA] * 4
        ),
    )
    def seg(h_hbm, ei_hbm, out_hbm, acc, qa, qb, ra, rb, ia, ib, ga, gb):
        c = lax.axis_index("c")
        s = lax.axis_index("s")
        slab_off = pl.multiple_of(s * SLAB, 8)

        @pl.when((c == 0) & (s < NS - 1))
        def _():
            pltpu.sync_copy(h_hbm.at[pl.ds(slab_off, SLAB)],
                            acc.at[pl.ds(slab_off, SLAB)])

        @pl.when((c == 0) & (s == NS - 1))
        def _():
            pltpu.sync_copy(h_hbm.at[pl.ds((NS - 1) * SLAB, LAST)],
                            acc.at[pl.ds((NS - 1) * SLAB, LAST)])

        plsc.subcore_barrier()

        @pl.when(c == 0)
        def _():
            base = s * cpt
            qs, isems = (qa, qb), (ia, ib)
            rows, gsems = (ra, rb), (ga, gb)

            pltpu.sync_copy(ei_hbm.at[pl.ds(base, 4)], qa)
            pltpu.async_copy(ei_hbm.at[pl.ds(base + 4, 4)], qb, ib)
            pltpu.async_copy(h_hbm.at[qa.at[0, 0]], ra, ga)
            pltpu.async_copy(h_hbm.at[qa.at[1, 0]], rb, gb)

            @pl.loop(0, cpt, step=8)
            def _(j):
                for k in range(8):
                    qk = qs[k // 4]
                    qn = qs[((k + 2) % 8) // 4]
                    rbuf, gsem = rows[k % 2], gsems[k % 2]
                    if k == 2:
                        pltpu.make_async_copy(ei_hbm.at[pl.ds(base, 4)], qb,
                                              ib).wait()
                    if k == 6:
                        pltpu.make_async_copy(ei_hbm.at[pl.ds(base, 4)], qa,
                                              ia).wait()
                    pltpu.make_async_copy(h_hbm.at[qk.at[k % 4, 0]], rbuf,
                                          gsem).wait()
                    pltpu.sync_copy(rbuf, acc.at[qk.at[k % 4, 1]], add=True)
                    pltpu.async_copy(h_hbm.at[qn.at[(k + 2) % 4, 0]], rbuf, gsem)
                    if k == 3:
                        pltpu.async_copy(ei_hbm.at[pl.ds(base + j + 8, 4)], qa, ia)
                    if k == 7:
                        pltpu.async_copy(ei_hbm.at[pl.ds(base + j + 12, 4)], qb, ib)

            pltpu.make_async_copy(h_hbm.at[qa.at[0, 0]], ra, ga).wait()
            pltpu.make_async_copy(h_hbm.at[qa.at[1, 0]], rb, gb).wait()
            pltpu.make_async_copy(ei_hbm.at[pl.ds(0, 4)], qb, ib).wait()

        plsc.subcore_barrier()

        @pl.when((c == 0) & (s < NS - 1))
        def _():
            pltpu.sync_copy(acc.at[pl.ds(slab_off, SLAB)],
                            out_hbm.at[pl.ds(slab_off, SLAB)])

        @pl.when((c == 0) & (s == NS - 1))
        def _():
            pltpu.sync_copy(acc.at[pl.ds((NS - 1) * SLAB, LAST)],
                            out_hbm.at[pl.ds((NS - 1) * SLAB, LAST)])

    return seg


def _segment_partials(h, ei, cpt):
    return _make_seg_kernel(cpt)(h, ei)


def _mm(a, w):
    return lax.dot_general(a, w, (((1,), (1,)), ((), ())),
                           preferred_element_type=jnp.float32,
                           precision=lax.Precision.HIGHEST)


def _mlp_bn(p_ref, x, wa_ref, ba_ref, wb_ref, bb_ref, g_ref, b_ref):
    y = p_ref[...]
    t = jnp.maximum(_mm(y, wa_ref[...]) + ba_ref[...], 0.0)
    z = _mm(t, wb_ref[...]) + bb_ref[...]
    m = jnp.mean(z, axis=0, keepdims=True)
    v = jnp.mean((z - m) ** 2, axis=0, keepdims=True)
    return (z - m) / jnp.sqrt(v + 1e-5) * g_ref[...] + b_ref[...]


def _l1_body(x_ref, p_ref, wa_ref, ba_ref, wb_ref, bb_ref, g_ref, b_ref,
             wr_ref, br_ref, out_ref):
    x = x_ref[...]
    zn = _mlp_bn(p_ref, x, wa_ref, ba_ref, wb_ref, bb_ref, g_ref, b_ref)
    res = _mm(x, wr_ref[...]) + br_ref[...]
    out_ref[...] = jnp.maximum(res + zn, 0.0)


def _l2_body(x_ref, p_ref, wa_ref, ba_ref, wb_ref, bb_ref, g_ref, b_ref,
             out_ref):
    x = x_ref[...]
    zn = _mlp_bn(p_ref, x, wa_ref, ba_ref, wb_ref, bb_ref, g_ref, b_ref)
    out_ref[...] = jnp.maximum(x + zn, 0.0)


def _l3_body(x_ref, p_ref, wa_ref, ba_ref, wb_ref, bb_ref, g_ref, b_ref,
             wr_ref, br_ref, out_ref):
    x = x_ref[...]
    zn = _mlp_bn(p_ref, x, wa_ref, ba_ref, wb_ref, bb_ref, g_ref, b_ref)
    u = _mm(x, wr_ref[...]) + br_ref[...] + zn
    mx = jnp.max(u, axis=1, keepdims=True)
    lse = jnp.log(jnp.sum(jnp.exp(u - mx), axis=1, keepdims=True)) + mx
    out_ref[...] = u - lse


_OUT = jax.ShapeDtypeStruct((N, D), jnp.float32)
_l1_call = pl.pallas_call(_l1_body, out_shape=_OUT)
_l2_call = pl.pallas_call(_l2_body, out_shape=_OUT)
_l3_call = pl.pallas_call(_l3_body, out_shape=_OUT)


def kernel(x, edge_index, w1a, b1a, w1b, b1b, w2a, b2a, w2b, b2b,
           w3a, b3a, w3b, b3b, bn1_g, bn1_b, bn2_g, bn2_b, wr1, br1, wr2, br2):
    src = edge_index[0]
    dst = edge_index[1]
    E = src.shape[0]
    cpt = -(-(-(-E // CHUNK)) // (NS * 8)) * 8
    nrows = NS * cpt
    pad = nrows * CHUNK - E
    srcp = jnp.concatenate([src, jnp.zeros((pad,), jnp.int32)]).reshape(nrows, CHUNK)
    dstp = jnp.concatenate([dst, jnp.full((pad,), N, jnp.int32)]).reshape(nrows, CHUNK)
    ei = jnp.stack([srcp, dstp], axis=1)
    tail = jnp.concatenate([jnp.zeros((8, 1, CHUNK), jnp.int32),
                            jnp.full((8, 1, CHUNK), N, jnp.int32)], axis=1)
    ei = jnp.concatenate([ei, tail], axis=0)

    r = lambda v: v.reshape(1, D)
    b1a_, b1b_, b2a_, b2b_, b3a_, b3b_ = map(r, (b1a, b1b, b2a, b2b, b3a, b3b))
    g1, bt1, g2, bt2 = map(r, (bn1_g, bn1_b, bn2_g, bn2_b))
    br1_, br2_ = r(br1), r(br2)

    p = _segment_partials(x, ei, cpt)
    h1 = _l1_call(x, p, w1a, b1a_, w1b, b1b_, g1, bt1, wr1, br1_)
    p = _segment_partials(h1, ei, cpt)
    h2 = _l2_call(h1, p, w2a, b2a_, w2b, b2b_, g1, bt1)
    p = _segment_partials(h2, ei, cpt)
    return _l3_call(h2, p, w3a, b3a_, w3b, b3b_, g2, bt2, wr2, br2_)
